# baseline (speedup 1.0000x reference)
import jax
import jax.numpy as jnp
from jax import lax
from jax.experimental import pallas as pl
from jax.experimental.pallas import tpu as pltpu
from jax.experimental.pallas import tpu_sc as plsc

F = 128
N = 10000
MAX_NEIGH = 33
LANES = 16
NC = 2
NS = 16
NW = NC * NS
F_PER_W = F // NW
JPACK = (MAX_NEIGH + 1) // 2
CHUNK = 400
NCHUNKS = N // CHUNK
GROUPS = CHUNK // LANES
NPAIRS = (NCHUNKS - 1) // 2


def _sc_body(x_hbm, nbc_hbm, w_hbm, out_hbm,
             x_t, xp_t, idx0, idx1, out0, out1, w_t,
             isem0, isem1, osem0, osem1):
    wid = lax.axis_index("s") * NC + lax.axis_index("c")
    f0 = wid * F_PER_W
    npairs_f = F_PER_W // 2

    idx_t = (idx0, idx1)
    out_t = (out0, out1)
    isem = (isem0, isem1)
    osem = (osem0, osem1)

    pltpu.async_copy(nbc_hbm.at[0], idx0, isem0)
    pltpu.async_copy(nbc_hbm.at[1], idx1, isem1)
    pltpu.sync_copy(x_hbm.at[pl.ds(f0, F_PER_W)], x_t)
    pltpu.sync_copy(w_hbm, w_t)
    w0 = w_t[0, :]
    w1 = w_t[1, :]

    def packrow(g, carry):
        base = g * LANES
        for p in range(npairs_f):
            a = x_t[2 * p, pl.ds(base, LANES)]
            b = x_t[2 * p + 1, pl.ds(base, LANES)]
            pk = plsc.pack(a, b, format=plsc.PackFormat.INTERLEAVED)
            xp_t[p, pl.ds(base, LANES)] = plsc.bitcast(pk, jnp.int32)
        return carry

    lax.fori_loop(0, N // LANES, packrow, 0, unroll=False)

    def compute_chunk(ci, ib, ob):
        c0 = ci * CHUNK

        def group(base, carry):
            words = [ib[j2, pl.ds(base, LANES)] for j2 in range(JPACK)]
            idxv = []
            for j2 in range(JPACK):
                idxv.append(words[j2] & 0xFFFF)
                if 2 * j2 + 1 < MAX_NEIGH:
                    idxv.append(lax.shift_right_logical(words[j2], 16))
            for p in range(npairs_f):
                pvec = jnp.full((LANES,), p, jnp.int32)

                def gat(j):
                    word = plsc.load_gather(xp_t, [pvec, idxv[j]])
                    return plsc.bitcast(word, jnp.bfloat16)

                acc_a = gat(0)
                acc_b = gat(1)
                for j in range(2, MAX_NEIGH):
                    if j % 2 == 0:
                        acc_a = jnp.maximum(acc_a, gat(j))
                    else:
                        acc_b = jnp.maximum(acc_b, gat(j))
                m_lo, m_hi = plsc.unpack(
                    jnp.maximum(acc_a, acc_b), format=plsc.PackFormat.INTERLEAVED
                )
                sw = xp_t[p, pl.ds(c0 + base, LANES)]
                xs_lo, xs_hi = plsc.unpack(
                    plsc.bitcast(sw, jnp.bfloat16), format=plsc.PackFormat.INTERLEAVED
                )
                for s, m, xs in ((0, m_lo, xs_lo), (1, m_hi, xs_hi)):
                    f = 2 * p + s
                    ob[f, pl.ds(base, LANES)] = w0 * xs + w1 * m
            return carry

        plsc.parallel_loop(0, CHUNK, LANES, unroll=4, carry=jnp.int32(0))(group)

    def store_chunk(ci, ob, sem):
        c0 = ci * CHUNK
        for f in range(F_PER_W):
            pltpu.async_copy(ob.at[f], out_hbm.at[f0 + f, pl.ds(c0, CHUNK)], sem)

    def drain_store(ob, sem):
        for f in range(F_PER_W):
            pltpu.make_async_copy(ob.at[f], out_hbm.at[f0 + f, pl.ds(0, CHUNK)], sem).wait()

    def pair(i, carry):
        for par in (0, 1):
            ci = 2 * i + par
            pltpu.make_async_copy(nbc_hbm.at[0], idx_t[par], isem[par]).wait()
            @pl.when(i >= 1)
            def _():
                drain_store(out_t[par], osem[par])
            compute_chunk(ci, idx_t[par], out_t[par])
            if par == 0:
                pltpu.async_copy(nbc_hbm.at[ci + 2], idx_t[0], isem[0])
            else:
                @pl.when(i < NPAIRS - 1)
                def _():
                    pltpu.async_copy(nbc_hbm.at[ci + 2], idx_t[1], isem[1])
            store_chunk(ci, out_t[par], osem[par])
        return carry

    lax.fori_loop(0, NPAIRS, pair, 0, unroll=False)

    ci = NCHUNKS - 1
    pltpu.make_async_copy(nbc_hbm.at[0], idx_t[0], isem[0]).wait()
    drain_store(out_t[0], osem[0])
    compute_chunk(ci, idx_t[0], out_t[0])
    store_chunk(ci, out_t[0], osem[0])
    drain_store(out_t[0], osem[0])
    drain_store(out_t[1], osem[1])


@jax.jit
def _run(x2, nbc, w2):
    mesh = plsc.VectorSubcoreMesh(
        core_axis_name="c", subcore_axis_name="s", num_cores=NC, num_subcores=NS
    )
    k = pl.kernel(
        _sc_body,
        out_type=jax.ShapeDtypeStruct((F, N), jnp.float32),
        mesh=mesh,
        scratch_types=[
            pltpu.VMEM((F_PER_W, N), jnp.float32),
            pltpu.VMEM((F_PER_W // 2, N), jnp.int32),
            pltpu.VMEM((JPACK, CHUNK), jnp.int32),
            pltpu.VMEM((JPACK, CHUNK), jnp.int32),
            pltpu.VMEM((F_PER_W, CHUNK), jnp.float32),
            pltpu.VMEM((F_PER_W, CHUNK), jnp.float32),
            pltpu.VMEM((2, LANES), jnp.float32),
            pltpu.SemaphoreType.DMA,
            pltpu.SemaphoreType.DMA,
            pltpu.SemaphoreType.DMA,
            pltpu.SemaphoreType.DMA,
        ],
        compiler_params=pltpu.CompilerParams(
            use_tc_tiling_on_sc=False, needs_layout_passes=False
        ),
    )
    return k(x2, nbc, w2)


def kernel(x, neighborhood, weight):
    x2 = x.reshape(F, N)
    nb34 = jnp.concatenate([neighborhood, neighborhood[:, -1:]], axis=1)
    nb17 = jax.lax.bitcast_convert_type(
        nb34.astype(jnp.uint16).reshape(N, JPACK, 2), jnp.int32
    )
    nbc = nb17.reshape(NCHUNKS, CHUNK, JPACK).transpose(0, 2, 1)
    w2 = jnp.broadcast_to(weight.reshape(2, 1), (2, LANES))
    out = _run(x2, nbc, w2)
    return out.reshape(1, F, N)


# speedup vs baseline: 1.3701x; 1.3701x over previous
import jax
import jax.numpy as jnp
from jax import lax
from jax.experimental import pallas as pl
from jax.experimental.pallas import tpu as pltpu
from jax.experimental.pallas import tpu_sc as plsc

F = 128
N = 10000
MAX_NEIGH = 33
LANES = 16
NC = 2
NS = 16
NW = NC * NS
F_PER_W = F // NW
JPACK = (MAX_NEIGH + 1) // 2
CHUNK = 400
NCHUNKS = N // CHUNK
GROUPS = CHUNK // LANES
NPAIRS = (NCHUNKS - 1) // 2


def _sc_body(x_hbm, nbc_hbm, w_hbm, out_hbm,
             x_t, xp_t, idx0, idx1, out0, out1, w_t,
             isem0, isem1, osem0, osem1):
    wid = lax.axis_index("s") * NC + lax.axis_index("c")
    f0 = wid * F_PER_W
    npairs_f = F_PER_W // 2

    idx_t = (idx0, idx1)
    out_t = (out0, out1)
    isem = (isem0, isem1)
    osem = (osem0, osem1)

    pltpu.async_copy(nbc_hbm.at[0], idx0, isem0)
    pltpu.async_copy(nbc_hbm.at[1], idx1, isem1)
    pltpu.sync_copy(x_hbm.at[pl.ds(f0, F_PER_W)], x_t)
    pltpu.sync_copy(w_hbm, w_t)
    w0 = w_t[0, :]
    w1 = w_t[1, :]

    def packrow(g, carry):
        base = g * LANES
        for p in range(npairs_f):
            a = x_t[2 * p, pl.ds(base, LANES)]
            b = x_t[2 * p + 1, pl.ds(base, LANES)]
            pk = plsc.pack(a, b, format=plsc.PackFormat.INTERLEAVED)
            xp_t[p, pl.ds(base, LANES)] = plsc.bitcast(pk, jnp.int32)
        return carry

    lax.fori_loop(0, N // LANES, packrow, 0, unroll=False)

    def compute_chunk(ci, ib, ob):
        c0 = ci * CHUNK

        def group(base, carry):
            words = [ib[j2, pl.ds(base, LANES)] for j2 in range(JPACK)]
            idxv = []
            for j2 in range(JPACK):
                idxv.append(words[j2] & 0xFFFF)
                if 2 * j2 + 1 < MAX_NEIGH:
                    idxv.append(lax.shift_right_logical(words[j2], 16))
            for p in range(npairs_f):
                pvec = jnp.full((LANES,), p, jnp.int32)

                def gat(j):
                    word = plsc.load_gather(xp_t, [pvec, idxv[j]])
                    return plsc.bitcast(word, jnp.bfloat16)

                acc_a = gat(0)
                acc_b = gat(1)
                for j in range(2, MAX_NEIGH):
                    if j % 2 == 0:
                        acc_a = jnp.maximum(acc_a, gat(j))
                    else:
                        acc_b = jnp.maximum(acc_b, gat(j))
                m_lo, m_hi = plsc.unpack(
                    jnp.maximum(acc_a, acc_b), format=plsc.PackFormat.INTERLEAVED
                )
                sw = xp_t[p, pl.ds(c0 + base, LANES)]
                xs_lo, xs_hi = plsc.unpack(
                    plsc.bitcast(sw, jnp.bfloat16), format=plsc.PackFormat.INTERLEAVED
                )
                for s, m, xs in ((0, m_lo, xs_lo), (1, m_hi, xs_hi)):
                    f = 2 * p + s
                    ob[f, pl.ds(base, LANES)] = w0 * xs + w1 * m
            return carry

        plsc.parallel_loop(0, CHUNK, LANES, unroll=1, carry=jnp.int32(0))(group)

    def store_chunk(ci, ob, sem):
        c0 = ci * CHUNK
        for f in range(F_PER_W):
            pltpu.async_copy(ob.at[f], out_hbm.at[f0 + f, pl.ds(c0, CHUNK)], sem)

    def drain_store(ob, sem):
        for f in range(F_PER_W):
            pltpu.make_async_copy(ob.at[f], out_hbm.at[f0 + f, pl.ds(0, CHUNK)], sem).wait()

    def pair(i, carry):
        for par in (0, 1):
            ci = 2 * i + par
            pltpu.make_async_copy(nbc_hbm.at[0], idx_t[par], isem[par]).wait()
            @pl.when(i >= 1)
            def _():
                drain_store(out_t[par], osem[par])
            compute_chunk(ci, idx_t[par], out_t[par])
            if par == 0:
                pltpu.async_copy(nbc_hbm.at[ci + 2], idx_t[0], isem[0])
            else:
                @pl.when(i < NPAIRS - 1)
                def _():
                    pltpu.async_copy(nbc_hbm.at[ci + 2], idx_t[1], isem[1])
            store_chunk(ci, out_t[par], osem[par])
        return carry

    lax.fori_loop(0, NPAIRS, pair, 0, unroll=False)

    ci = NCHUNKS - 1
    pltpu.make_async_copy(nbc_hbm.at[0], idx_t[0], isem[0]).wait()
    drain_store(out_t[0], osem[0])
    compute_chunk(ci, idx_t[0], out_t[0])
    store_chunk(ci, out_t[0], osem[0])
    drain_store(out_t[0], osem[0])
    drain_store(out_t[1], osem[1])


@jax.jit
def _run(x2, nbc, w2):
    mesh = plsc.VectorSubcoreMesh(
        core_axis_name="c", subcore_axis_name="s", num_cores=NC, num_subcores=NS
    )
    k = pl.kernel(
        _sc_body,
        out_type=jax.ShapeDtypeStruct((F, N), jnp.float32),
        mesh=mesh,
        scratch_types=[
            pltpu.VMEM((F_PER_W, N), jnp.float32),
            pltpu.VMEM((F_PER_W // 2, N), jnp.int32),
            pltpu.VMEM((JPACK, CHUNK), jnp.int32),
            pltpu.VMEM((JPACK, CHUNK), jnp.int32),
            pltpu.VMEM((F_PER_W, CHUNK), jnp.float32),
            pltpu.VMEM((F_PER_W, CHUNK), jnp.float32),
            pltpu.VMEM((2, LANES), jnp.float32),
            pltpu.SemaphoreType.DMA,
            pltpu.SemaphoreType.DMA,
            pltpu.SemaphoreType.DMA,
            pltpu.SemaphoreType.DMA,
        ],
        compiler_params=pltpu.CompilerParams(
            use_tc_tiling_on_sc=False, needs_layout_passes=False
        ),
    )
    return k(x2, nbc, w2)


def kernel(x, neighborhood, weight):
    x2 = x.reshape(F, N)
    nb34 = jnp.concatenate([neighborhood, neighborhood[:, -1:]], axis=1)
    nb17 = jax.lax.bitcast_convert_type(
        nb34.astype(jnp.uint16).reshape(N, JPACK, 2), jnp.int32
    )
    nbc = nb17.reshape(NCHUNKS, CHUNK, JPACK).transpose(0, 2, 1)
    w2 = jnp.broadcast_to(weight.reshape(2, 1), (2, LANES))
    out = _run(x2, nbc, w2)
    return out.reshape(1, F, N)
